# baseline (speedup 1.0000x reference)
import jax
import jax.numpy as jnp
from jax import lax
from jax.experimental import pallas as pl
from jax.experimental.pallas import tpu as pltpu
from jax.experimental.pallas import tpu_sc as plsc

_VOCAB = 100000
_DIM = 64
_B = 16384
_L = 10
_NEG = 5
_T = _NEG + 1
_NC = 2
_NS = 16
_NW = _NC * _NS
_BPW = _B // _NW
_CH = 64
_NCHUNK = _BPW // _CH
_LANES = 16
_DC = _DIM // _LANES
_GROUPS = _CH // _LANES
_Y2COLS = 256
_Y2ROWS = _T * _B * _LANES // _Y2COLS
_BROWS = _B * _LANES // _Y2COLS
_CIDX = _CH * _L
_NIDX = _CH * _NEG


def _cbow_sc_body(ctx_hbm, word_hbm, neg_hbm, emb0_hbm, emb1_hbm, y_hbm,
                  ctx2d, neg2d, word_st, ctx_idx, neg_idx,
                  ctx_rows, word_rows, neg_rows, y_v, sem_g):
    wid = lax.axis_index("s") * _NC + lax.axis_index("c")
    base = wid * _BPW

    def chunk_body(c, carry):
        cb = base + c * _CH
        pltpu.sync_copy(
            ctx_hbm.at[pl.ds(pl.multiple_of(cb * 128, 8), _CH * 128)], ctx2d)
        pltpu.sync_copy(
            neg_hbm.at[pl.ds(pl.multiple_of(cb * 128, 8), _CH * 128)], neg2d)
        pltpu.sync_copy(word_hbm.at[pl.ds(pl.multiple_of(cb, 8), _CH)],
                        word_st)
        for e in range(_CH):
            ctx_idx[pl.ds(e * _L, _LANES)] = ctx2d[pl.ds(e * 128, _LANES)]
            neg_idx[pl.ds(e * _NEG, _LANES)] = neg2d[pl.ds(e * 128, _LANES)]
        handles = []
        for j in range(_CIDX // 128):
            handles.append(pltpu.async_copy(
                emb0_hbm.at[ctx_idx.at[pl.ds(j * 128, 128)]],
                ctx_rows.at[pl.ds(j * 128, 128)], sem_g))
        handles.append(pltpu.async_copy(
            emb1_hbm.at[word_st], word_rows, sem_g))
        for j in range(_NIDX // 64):
            handles.append(pltpu.async_copy(
                emb1_hbm.at[neg_idx.at[pl.ds(j * 64, 64)]],
                neg_rows.at[pl.ds(j * 64, 64)], sem_g))
        for h in handles:
            h.wait()

        def group(g, carry2):
            for p in range(_LANES):
                e = g * _LANES + p
                csum = []
                for k in range(_DC):
                    s = ctx_rows[e * _L, pl.ds(k * _LANES, _LANES)]
                    for j in range(1, _L):
                        s = s + ctx_rows[e * _L + j,
                                         pl.ds(k * _LANES, _LANES)]
                    csum.append(s)
                v = csum[0] * word_rows[e, pl.ds(0, _LANES)]
                for k in range(1, _DC):
                    v = v + csum[k] * word_rows[e, pl.ds(k * _LANES, _LANES)]
                y_v[0, g, pl.ds(p * _LANES, _LANES)] = v
                for r in range(_NEG):
                    v = csum[0] * neg_rows[e * _NEG + r, pl.ds(0, _LANES)]
                    for k in range(1, _DC):
                        v = v + csum[k] * neg_rows[e * _NEG + r,
                                                   pl.ds(k * _LANES, _LANES)]
                    y_v[1 + r, g, pl.ds(p * _LANES, _LANES)] = v
            return carry2

        lax.fori_loop(0, _GROUPS, group, 0)
        for r in range(_T):
            pltpu.sync_copy(
                y_v.at[r],
                y_hbm.at[pl.ds(r * _BROWS + (cb // _LANES), _GROUPS)])
        return carry

    lax.fori_loop(0, _NCHUNK, chunk_body, 0)


def _loss_tc_body(y2_ref, lens_ref, o_ref):
    y2 = y2_ref[...]
    seg = (lax.broadcasted_iota(jnp.int32, (_Y2COLS, _LANES), 0) // _LANES
           == lax.broadcasted_iota(jnp.int32, (_Y2COLS, _LANES), 1))
    x = jnp.dot(y2, seg.astype(jnp.float32),
                preferred_element_type=jnp.float32)
    x3 = x.reshape(_T, _BROWS, _LANES) / lens_ref[...][None, :, :]
    sgn = jnp.where(
        lax.broadcasted_iota(jnp.int32, (_T, _BROWS, _LANES), 0) == 0,
        1.0, -1.0)
    terms = -jax.nn.log_sigmoid(sgn * jnp.clip(x3, -10.0, 10.0))
    o_ref[...] = jnp.sum(terms)[None, None]


@jax.jit
def _cbow(ctx_pad, word_idx, neg_pad, lens2, emb0_weight, emb1_weight):
    mesh = plsc.VectorSubcoreMesh(core_axis_name="c", subcore_axis_name="s")
    y = pl.kernel(
        _cbow_sc_body,
        mesh=mesh,
        compiler_params=pltpu.CompilerParams(use_tc_tiling_on_sc=False),
        out_type=jax.ShapeDtypeStruct((_Y2ROWS, _Y2COLS), jnp.float32),
        scratch_types=[
            pltpu.VMEM((_CH * 128,), jnp.int32),
            pltpu.VMEM((_CH * 128,), jnp.int32),
            pltpu.VMEM((_CH,), jnp.int32),
            pltpu.VMEM((_CIDX + _LANES,), jnp.int32),
            pltpu.VMEM((_NIDX + _LANES,), jnp.int32),
            pltpu.VMEM((_CIDX, _DIM), jnp.float32),
            pltpu.VMEM((_CH, _DIM), jnp.float32),
            pltpu.VMEM((_NIDX, _DIM), jnp.float32),
            pltpu.VMEM((_T, _GROUPS, _Y2COLS), jnp.float32),
            pltpu.SemaphoreType.DMA,
        ],
    )(ctx_pad, word_idx, neg_pad, emb0_weight, emb1_weight)
    o = pl.pallas_call(
        _loss_tc_body,
        out_shape=jax.ShapeDtypeStruct((1, 1), jnp.float32),
    )(y, lens2)
    return o[0, 0]


def kernel(word_idx, ctx_inds, ctx_lens, neg_inds, emb0_weight, emb1_weight):
    ctx_pad = jnp.pad(ctx_inds.astype(jnp.int32),
                      ((0, 0), (0, 128 - _L))).reshape(-1)
    neg_pad = jnp.pad(neg_inds.astype(jnp.int32),
                      ((0, 0), (0, 128 - _NEG))).reshape(-1)
    lens2 = ctx_lens.astype(jnp.float32).reshape(_BROWS, _LANES)
    return _cbow(ctx_pad, word_idx.astype(jnp.int32), neg_pad, lens2,
                 emb0_weight, emb1_weight)


# speedup vs baseline: 1.0617x; 1.0617x over previous
import jax
import jax.numpy as jnp
from jax import lax
from jax.experimental import pallas as pl
from jax.experimental.pallas import tpu as pltpu
from jax.experimental.pallas import tpu_sc as plsc

_VOCAB = 100000
_DIM = 64
_B = 16384
_L = 10
_NEG = 5
_T = _NEG + 1
_NC = 2
_NS = 16
_NW = _NC * _NS
_BPW = _B // _NW
_CH = 32
_NCHUNK = _BPW // _CH
_LANES = 16
_DC = _DIM // _LANES
_GROUPS = _CH // _LANES
_Y2COLS = 256
_Y2ROWS = _T * _B * _LANES // _Y2COLS
_BROWS = _B * _LANES // _Y2COLS
_CC = _CH * _L
_NN = _CH * _NEG
_CTX_BLK = [(0, 128), (128, 128), (256, 64)]
_NEG_BLK = [(0, 128), (128, 32)]


def _cbow_sc_body(ctx_hbm, word_hbm, neg_hbm, emb0_hbm, emb1_hbm, y_hbm,
                  ctx_idx, neg_idx, word_st,
                  ctx_rA, word_rA, neg_rA, ctx_rB, word_rB, neg_rB,
                  y_v, semA, semB):
    wid = lax.axis_index("s") * _NC + lax.axis_index("c")
    base = wid * _BPW

    pltpu.sync_copy(
        ctx_hbm.at[pl.ds(pl.multiple_of(base * _L, 8), _BPW * _L)], ctx_idx)
    pltpu.sync_copy(
        neg_hbm.at[pl.ds(pl.multiple_of(base * _NEG, 8), _BPW * _NEG)],
        neg_idx)
    pltpu.sync_copy(word_hbm.at[pl.ds(pl.multiple_of(base, 8), _BPW)],
                    word_st)

    def _descs(c, bufs, sem):
        ctx_r, word_r, neg_r = bufs
        out = []
        for off, sz in _CTX_BLK:
            out.append((emb0_hbm.at[ctx_idx.at[pl.ds(c * _CC + off, sz)]],
                        ctx_r.at[pl.ds(off, sz)], sem))
        out.append((emb1_hbm.at[word_st.at[pl.ds(c * _CH, _CH)]],
                    word_r, sem))
        for off, sz in _NEG_BLK:
            out.append((emb1_hbm.at[neg_idx.at[pl.ds(c * _NN + off, sz)]],
                        neg_r.at[pl.ds(off, sz)], sem))
        return out

    def fire(c, bufs, sem):
        for src, dst, sm in _descs(c, bufs, sem):
            pltpu.async_copy(src, dst, sm)

    def drain(c, bufs, sem):
        for src, dst, sm in _descs(c, bufs, sem):
            pltpu.make_async_copy(src, dst, sm).wait()

    def compute(c, bufs):
        ctx_r, word_r, neg_r = bufs
        cb = base + c * _CH

        def group(g, carry2):
            for p in range(_LANES):
                e = g * _LANES + p
                csum = []
                for k in range(_DC):
                    s = ctx_r[e * _L, pl.ds(k * _LANES, _LANES)]
                    for j in range(1, _L):
                        s = s + ctx_r[e * _L + j, pl.ds(k * _LANES, _LANES)]
                    csum.append(s)
                v = csum[0] * word_r[e, pl.ds(0, _LANES)]
                for k in range(1, _DC):
                    v = v + csum[k] * word_r[e, pl.ds(k * _LANES, _LANES)]
                y_v[0, g, pl.ds(p * _LANES, _LANES)] = v
                for r in range(_NEG):
                    v = csum[0] * neg_r[e * _NEG + r, pl.ds(0, _LANES)]
                    for k in range(1, _DC):
                        v = v + csum[k] * neg_r[e * _NEG + r,
                                                pl.ds(k * _LANES, _LANES)]
                    y_v[1 + r, g, pl.ds(p * _LANES, _LANES)] = v
            return carry2

        lax.fori_loop(0, _GROUPS, group, 0)
        for r in range(_T):
            pltpu.sync_copy(
                y_v.at[r],
                y_hbm.at[pl.ds(r * _BROWS + (cb // _LANES), _GROUPS)])

    bufsA = (ctx_rA, word_rA, neg_rA)
    bufsB = (ctx_rB, word_rB, neg_rB)

    fire(0, bufsA, semA)

    def pipe(c0, carry):
        fire(c0 + 1, bufsB, semB)
        drain(c0, bufsA, semA)
        compute(c0, bufsA)

        @pl.when(c0 + 2 < _NCHUNK)
        def _():
            fire(c0 + 2, bufsA, semA)

        drain(c0 + 1, bufsB, semB)
        compute(c0 + 1, bufsB)
        return carry

    lax.fori_loop(0, _NCHUNK // 2, lambda i, cr: pipe(i * 2, cr), 0)


def _loss_tc_body(y2_ref, lens_ref, o_ref):
    y2 = y2_ref[...]
    seg = (lax.broadcasted_iota(jnp.int32, (_Y2COLS, _LANES), 0) // _LANES
           == lax.broadcasted_iota(jnp.int32, (_Y2COLS, _LANES), 1))
    x = jnp.dot(y2, seg.astype(jnp.float32),
                preferred_element_type=jnp.float32)
    x3 = x.reshape(_T, _BROWS, _LANES) / lens_ref[...][None, :, :]
    sgn = jnp.where(
        lax.broadcasted_iota(jnp.int32, (_T, _BROWS, _LANES), 0) == 0,
        1.0, -1.0)
    terms = -jax.nn.log_sigmoid(sgn * jnp.clip(x3, -10.0, 10.0))
    o_ref[...] = jnp.sum(terms)[None, None]


@jax.jit
def _cbow(ctx_inds, word_idx, neg_inds, lens2, emb0_weight, emb1_weight):
    mesh = plsc.VectorSubcoreMesh(core_axis_name="c", subcore_axis_name="s")
    y = pl.kernel(
        _cbow_sc_body,
        mesh=mesh,
        compiler_params=pltpu.CompilerParams(use_tc_tiling_on_sc=False),
        out_type=jax.ShapeDtypeStruct((_Y2ROWS, _Y2COLS), jnp.float32),
        scratch_types=[
            pltpu.VMEM((_BPW * _L,), jnp.int32),
            pltpu.VMEM((_BPW * _NEG,), jnp.int32),
            pltpu.VMEM((_BPW,), jnp.int32),
            pltpu.VMEM((_CC, _DIM), jnp.float32),
            pltpu.VMEM((_CH, _DIM), jnp.float32),
            pltpu.VMEM((_NN, _DIM), jnp.float32),
            pltpu.VMEM((_CC, _DIM), jnp.float32),
            pltpu.VMEM((_CH, _DIM), jnp.float32),
            pltpu.VMEM((_NN, _DIM), jnp.float32),
            pltpu.VMEM((_T, _GROUPS, _Y2COLS), jnp.float32),
            pltpu.SemaphoreType.DMA,
            pltpu.SemaphoreType.DMA,
        ],
    )(ctx_inds, word_idx, neg_inds, emb0_weight, emb1_weight)
    o = pl.pallas_call(
        _loss_tc_body,
        out_shape=jax.ShapeDtypeStruct((1, 1), jnp.float32),
    )(y, lens2)
    return o[0, 0]


def kernel(word_idx, ctx_inds, ctx_lens, neg_inds, emb0_weight, emb1_weight):
    lens2 = ctx_lens.astype(jnp.float32).reshape(_BROWS, _LANES)
    return _cbow(ctx_inds.astype(jnp.int32).reshape(-1),
                 word_idx.astype(jnp.int32),
                 neg_inds.astype(jnp.int32).reshape(-1), lens2,
                 emb0_weight, emb1_weight)


# speedup vs baseline: 1.0816x; 1.0188x over previous
import jax
import jax.numpy as jnp
from jax import lax
from jax.experimental import pallas as pl
from jax.experimental.pallas import tpu as pltpu
from jax.experimental.pallas import tpu_sc as plsc

_VOCAB = 100000
_DIM = 64
_B = 16384
_L = 10
_NEG = 5
_T = _NEG + 1
_NC = 2
_NS = 16
_NW = _NC * _NS
_BPW = _B // _NW
_CH = 32
_NCHUNK = _BPW // _CH
_LANES = 16
_DC = _DIM // _LANES
_GROUPS = _CH // _LANES
_Y2COLS = 256
_Y2ROWS = _T * _B * _LANES // _Y2COLS
_BROWS = _B * _LANES // _Y2COLS
_CC = _CH * _L
_NN = _CH * _NEG
_CTX_BLK = [(0, 128), (128, 128), (256, 64)]
_NEG_BLK = [(0, 128), (128, 32)]


def _cbow_sc_body(ctx_hbm, word_hbm, neg_hbm, emb0_hbm, emb1_hbm, y_hbm,
                  ctx_idx, neg_idx, word_st,
                  ctx_rA, word_rA, neg_rA, ctx_rB, word_rB, neg_rB,
                  y_v, semA, semB):
    wid = lax.axis_index("s") * _NC + lax.axis_index("c")
    base = wid * _BPW

    pltpu.sync_copy(
        ctx_hbm.at[pl.ds(pl.multiple_of(base * _L, 8), _BPW * _L)], ctx_idx)
    pltpu.sync_copy(
        neg_hbm.at[pl.ds(pl.multiple_of(base * _NEG, 8), _BPW * _NEG)],
        neg_idx)
    pltpu.sync_copy(word_hbm.at[pl.ds(pl.multiple_of(base, 8), _BPW)],
                    word_st)

    def _descs(c, bufs, sem):
        ctx_r, word_r, neg_r = bufs
        out = []
        for off, sz in _CTX_BLK:
            out.append((emb0_hbm.at[ctx_idx.at[pl.ds(c * _CC + off, sz)]],
                        ctx_r.at[pl.ds(off, sz)], sem))
        out.append((emb1_hbm.at[word_st.at[pl.ds(c * _CH, _CH)]],
                    word_r, sem))
        for off, sz in _NEG_BLK:
            out.append((emb1_hbm.at[neg_idx.at[pl.ds(c * _NN + off, sz)]],
                        neg_r.at[pl.ds(off, sz)], sem))
        return out

    def fire(c, bufs, sem):
        for src, dst, sm in _descs(c, bufs, sem):
            pltpu.async_copy(src, dst, sm)

    def drain(c, bufs, sem):
        for src, dst, sm in _descs(c, bufs, sem):
            pltpu.make_async_copy(src, dst, sm).wait()

    def compute(c, bufs):
        ctx_r, word_r, neg_r = bufs

        def group(g, carry2):
            row = c * _GROUPS + g
            for p in range(_LANES):
                e = g * _LANES + p
                csum = []
                for k in range(_DC):
                    s = ctx_r[e * _L, pl.ds(k * _LANES, _LANES)]
                    for j in range(1, _L):
                        s = s + ctx_r[e * _L + j, pl.ds(k * _LANES, _LANES)]
                    csum.append(s)
                v = csum[0] * word_r[e, pl.ds(0, _LANES)]
                for k in range(1, _DC):
                    v = v + csum[k] * word_r[e, pl.ds(k * _LANES, _LANES)]
                y_v[0, row, pl.ds(p * _LANES, _LANES)] = v
                for r in range(_NEG):
                    v = csum[0] * neg_r[e * _NEG + r, pl.ds(0, _LANES)]
                    for k in range(1, _DC):
                        v = v + csum[k] * neg_r[e * _NEG + r,
                                                pl.ds(k * _LANES, _LANES)]
                    y_v[1 + r, row, pl.ds(p * _LANES, _LANES)] = v
            return carry2

        lax.fori_loop(0, _GROUPS, group, 0)

    bufsA = (ctx_rA, word_rA, neg_rA)
    bufsB = (ctx_rB, word_rB, neg_rB)

    fire(0, bufsA, semA)

    def pipe(c0, carry):
        fire(c0 + 1, bufsB, semB)
        drain(c0, bufsA, semA)
        compute(c0, bufsA)

        @pl.when(c0 + 2 < _NCHUNK)
        def _():
            fire(c0 + 2, bufsA, semA)

        drain(c0 + 1, bufsB, semB)
        compute(c0 + 1, bufsB)
        return carry

    lax.fori_loop(0, _NCHUNK // 2, lambda i, cr: pipe(i * 2, cr), 0)

    wrows = _BPW * _LANES // _Y2COLS
    for r in range(_T):
        pltpu.sync_copy(
            y_v.at[r],
            y_hbm.at[pl.ds(r * _BROWS + (base // _LANES), wrows)])


def _loss_tc_body(y2_ref, lens_ref, o_ref):
    y2 = y2_ref[...]
    seg = (lax.broadcasted_iota(jnp.int32, (_Y2COLS, _LANES), 0) // _LANES
           == lax.broadcasted_iota(jnp.int32, (_Y2COLS, _LANES), 1))
    x = jnp.dot(y2, seg.astype(jnp.float32),
                preferred_element_type=jnp.float32)
    x3 = x.reshape(_T, _BROWS, _LANES) / lens_ref[...][None, :, :]
    sgn = jnp.where(
        lax.broadcasted_iota(jnp.int32, (_T, _BROWS, _LANES), 0) == 0,
        1.0, -1.0)
    terms = -jax.nn.log_sigmoid(sgn * jnp.clip(x3, -10.0, 10.0))
    o_ref[...] = jnp.sum(terms)[None, None]


@jax.jit
def _cbow(ctx_inds, word_idx, neg_inds, lens2, emb0_weight, emb1_weight):
    mesh = plsc.VectorSubcoreMesh(core_axis_name="c", subcore_axis_name="s")
    y = pl.kernel(
        _cbow_sc_body,
        mesh=mesh,
        compiler_params=pltpu.CompilerParams(use_tc_tiling_on_sc=False),
        out_type=jax.ShapeDtypeStruct((_Y2ROWS, _Y2COLS), jnp.float32),
        scratch_types=[
            pltpu.VMEM((_BPW * _L,), jnp.int32),
            pltpu.VMEM((_BPW * _NEG,), jnp.int32),
            pltpu.VMEM((_BPW,), jnp.int32),
            pltpu.VMEM((_CC, _DIM), jnp.float32),
            pltpu.VMEM((_CH, _DIM), jnp.float32),
            pltpu.VMEM((_NN, _DIM), jnp.float32),
            pltpu.VMEM((_CC, _DIM), jnp.float32),
            pltpu.VMEM((_CH, _DIM), jnp.float32),
            pltpu.VMEM((_NN, _DIM), jnp.float32),
            pltpu.VMEM((_T, _BPW * _LANES // _Y2COLS, _Y2COLS), jnp.float32),
            pltpu.SemaphoreType.DMA,
            pltpu.SemaphoreType.DMA,
        ],
    )(ctx_inds, word_idx, neg_inds, emb0_weight, emb1_weight)
    o = pl.pallas_call(
        _loss_tc_body,
        out_shape=jax.ShapeDtypeStruct((1, 1), jnp.float32),
    )(y, lens2)
    return o[0, 0]


def kernel(word_idx, ctx_inds, ctx_lens, neg_inds, emb0_weight, emb1_weight):
    lens2 = ctx_lens.astype(jnp.float32).reshape(_BROWS, _LANES)
    return _cbow(ctx_inds.astype(jnp.int32).reshape(-1),
                 word_idx.astype(jnp.int32),
                 neg_inds.astype(jnp.int32).reshape(-1), lens2,
                 emb0_weight, emb1_weight)


# speedup vs baseline: 1.0942x; 1.0116x over previous
import jax
import jax.numpy as jnp
from jax import lax
from jax.experimental import pallas as pl
from jax.experimental.pallas import tpu as pltpu
from jax.experimental.pallas import tpu_sc as plsc

_VOCAB = 100000
_DIM = 64
_B = 16384
_L = 10
_NEG = 5
_T = _NEG + 1
_NC = 2
_NS = 16
_NW = _NC * _NS
_BPW = _B // _NW
_CH = 64
_NCHUNK = _BPW // _CH
_LANES = 16
_DC = _DIM // _LANES
_GROUPS = _CH // _LANES
_Y2COLS = 256
_Y2ROWS = _T * _B * _LANES // _Y2COLS
_BROWS = _B * _LANES // _Y2COLS
_CC = _CH * _L
_NN = _CH * _NEG
_CTX_BLK = [(0, 128), (128, 128), (256, 128), (384, 128), (512, 128)]
_NEG_BLK = [(0, 128), (128, 128), (256, 64)]


def _cbow_sc_body(ctx_hbm, word_hbm, neg_hbm, emb0_hbm, emb1_hbm, y_hbm,
                  ctx_idx, neg_idx, word_st,
                  ctx_rA, word_rA, neg_rA, y_v, semA):
    wid = lax.axis_index("s") * _NC + lax.axis_index("c")
    base = wid * _BPW

    pltpu.sync_copy(
        ctx_hbm.at[pl.ds(pl.multiple_of(base * _L, 8), _BPW * _L)], ctx_idx)
    pltpu.sync_copy(
        neg_hbm.at[pl.ds(pl.multiple_of(base * _NEG, 8), _BPW * _NEG)],
        neg_idx)
    pltpu.sync_copy(word_hbm.at[pl.ds(pl.multiple_of(base, 8), _BPW)],
                    word_st)

    def _descs(c, bufs, sem):
        ctx_r, word_r, neg_r = bufs
        out = []
        for off, sz in _CTX_BLK:
            out.append((emb0_hbm.at[ctx_idx.at[pl.ds(c * _CC + off, sz)]],
                        ctx_r.at[pl.ds(off, sz)], sem))
        out.append((emb1_hbm.at[word_st.at[pl.ds(c * _CH, _CH)]],
                    word_r, sem))
        for off, sz in _NEG_BLK:
            out.append((emb1_hbm.at[neg_idx.at[pl.ds(c * _NN + off, sz)]],
                        neg_r.at[pl.ds(off, sz)], sem))
        return out

    def fire(c, bufs, sem):
        for src, dst, sm in _descs(c, bufs, sem):
            pltpu.async_copy(src, dst, sm)

    def drain(c, bufs, sem):
        for src, dst, sm in _descs(c, bufs, sem):
            pltpu.make_async_copy(src, dst, sm).wait()

    def compute(c, bufs):
        ctx_r, word_r, neg_r = bufs

        def group(g, carry2):
            row = c * _GROUPS + g
            for p in range(_LANES):
                e = g * _LANES + p
                csum = []
                for k in range(_DC):
                    s = ctx_r[e * _L, pl.ds(k * _LANES, _LANES)]
                    for j in range(1, _L):
                        s = s + ctx_r[e * _L + j, pl.ds(k * _LANES, _LANES)]
                    csum.append(s)
                v = csum[0] * word_r[e, pl.ds(0, _LANES)]
                for k in range(1, _DC):
                    v = v + csum[k] * word_r[e, pl.ds(k * _LANES, _LANES)]
                y_v[0, row, pl.ds(p * _LANES, _LANES)] = v
                for r in range(_NEG):
                    v = csum[0] * neg_r[e * _NEG + r, pl.ds(0, _LANES)]
                    for k in range(1, _DC):
                        v = v + csum[k] * neg_r[e * _NEG + r,
                                                pl.ds(k * _LANES, _LANES)]
                    y_v[1 + r, row, pl.ds(p * _LANES, _LANES)] = v
            return carry2

        lax.fori_loop(0, _GROUPS, group, 0)

    bufsA = (ctx_rA, word_rA, neg_rA)

    def step(c, carry):
        fire(c, bufsA, semA)
        drain(c, bufsA, semA)
        compute(c, bufsA)
        return carry

    lax.fori_loop(0, _NCHUNK, step, 0)

    wrows = _BPW * _LANES // _Y2COLS
    for r in range(_T):
        pltpu.sync_copy(
            y_v.at[r],
            y_hbm.at[pl.ds(r * _BROWS + (base // _LANES), wrows)])


def _loss_tc_body(y2_ref, lens_ref, o_ref):
    y2 = y2_ref[...]
    seg = (lax.broadcasted_iota(jnp.int32, (_Y2COLS, _LANES), 0) // _LANES
           == lax.broadcasted_iota(jnp.int32, (_Y2COLS, _LANES), 1))
    x = jnp.dot(y2, seg.astype(jnp.float32),
                preferred_element_type=jnp.float32)
    x3 = x.reshape(_T, _BROWS, _LANES) / lens_ref[...][None, :, :]
    sgn = jnp.where(
        lax.broadcasted_iota(jnp.int32, (_T, _BROWS, _LANES), 0) == 0,
        1.0, -1.0)
    terms = -jax.nn.log_sigmoid(sgn * jnp.clip(x3, -10.0, 10.0))
    o_ref[...] = jnp.sum(terms)[None, None]


@jax.jit
def _cbow(ctx_inds, word_idx, neg_inds, lens2, emb0_weight, emb1_weight):
    mesh = plsc.VectorSubcoreMesh(core_axis_name="c", subcore_axis_name="s")
    y = pl.kernel(
        _cbow_sc_body,
        mesh=mesh,
        compiler_params=pltpu.CompilerParams(use_tc_tiling_on_sc=False),
        out_type=jax.ShapeDtypeStruct((_Y2ROWS, _Y2COLS), jnp.float32),
        scratch_types=[
            pltpu.VMEM((_BPW * _L,), jnp.int32),
            pltpu.VMEM((_BPW * _NEG,), jnp.int32),
            pltpu.VMEM((_BPW,), jnp.int32),
            pltpu.VMEM((_CC, _DIM), jnp.float32),
            pltpu.VMEM((_CH, _DIM), jnp.float32),
            pltpu.VMEM((_NN, _DIM), jnp.float32),
            pltpu.VMEM((_T, _BPW * _LANES // _Y2COLS, _Y2COLS), jnp.float32),
            pltpu.SemaphoreType.DMA,
        ],
    )(ctx_inds, word_idx, neg_inds, emb0_weight, emb1_weight)
    o = pl.pallas_call(
        _loss_tc_body,
        out_shape=jax.ShapeDtypeStruct((1, 1), jnp.float32),
    )(y, lens2)
    return o[0, 0]


def kernel(word_idx, ctx_inds, ctx_lens, neg_inds, emb0_weight, emb1_weight):
    lens2 = ctx_lens.astype(jnp.float32).reshape(_BROWS, _LANES)
    return _cbow(ctx_inds.astype(jnp.int32).reshape(-1),
                 word_idx.astype(jnp.int32),
                 neg_inds.astype(jnp.int32).reshape(-1), lens2,
                 emb0_weight, emb1_weight)


# speedup vs baseline: 1.2315x; 1.1255x over previous
import jax
import jax.numpy as jnp
from jax import lax
from jax.experimental import pallas as pl
from jax.experimental.pallas import tpu as pltpu
from jax.experimental.pallas import tpu_sc as plsc

_VOCAB = 100000
_DIM = 64
_B = 16384
_L = 10
_NEG = 5
_T = _NEG + 1
_NC = 2
_NS = 16
_NW = _NC * _NS
_BPW = _B // _NW
_CH = 64
_NCHUNK = _BPW // _CH
_LANES = 16
_DC = _DIM // _LANES
_YROWS = _T * _B
_Y2COLS = 256
_Y2ROWS = _YROWS * _LANES // _Y2COLS
_BROWS = _B * _LANES // _Y2COLS


def _cbow_sc_body(ctx_hbm, word_hbm, neg_hbm, emb0_hbm, emb1_hbm, y_hbm,
                  ctx_idx, neg_idx, word_st, ctx_rows, word_rows, neg_rows,
                  y_v, sem_g):
    wid = lax.axis_index("s") * _NC + lax.axis_index("c")
    base = wid * _BPW

    pltpu.sync_copy(
        ctx_hbm.at[pl.ds(pl.multiple_of(base * _L, 8), _BPW * _L)], ctx_idx)
    pltpu.sync_copy(
        neg_hbm.at[pl.ds(pl.multiple_of(base * _NEG, 8), _BPW * _NEG)],
        neg_idx)
    pltpu.sync_copy(word_hbm.at[pl.ds(pl.multiple_of(base, 8), _BPW)],
                    word_st)

    def chunk_body(c, carry):
        cb = base + c * _CH
        handles = []
        for j in range(_CH * _L // 128):
            handles.append(pltpu.async_copy(
                emb0_hbm.at[ctx_idx.at[pl.ds(c * _CH * _L + j * 128, 128)]],
                ctx_rows.at[pl.ds(j * 128, 128)], sem_g))
        handles.append(pltpu.async_copy(
            emb1_hbm.at[word_st.at[pl.ds(c * _CH, _CH)]], word_rows, sem_g))
        for j in range(_CH * _NEG // 64):
            handles.append(pltpu.async_copy(
                emb1_hbm.at[neg_idx.at[pl.ds(c * _CH * _NEG + j * 64, 64)]],
                neg_rows.at[pl.ds(j * 64, 64)], sem_g))
        for h in handles:
            h.wait()

        def elem(e, carry2):
            csum = []
            for k in range(_DC):
                s = ctx_rows[e * _L, pl.ds(k * _LANES, _LANES)]
                for j in range(1, _L):
                    s = s + ctx_rows[e * _L + j, pl.ds(k * _LANES, _LANES)]
                csum.append(s)
            v = csum[0] * word_rows[e, pl.ds(0, _LANES)]
            for k in range(1, _DC):
                v = v + csum[k] * word_rows[e, pl.ds(k * _LANES, _LANES)]
            y_v[0, e, pl.ds(0, _LANES)] = v
            for r in range(_NEG):
                v = csum[0] * neg_rows[e * _NEG + r, pl.ds(0, _LANES)]
                for k in range(1, _DC):
                    v = v + csum[k] * neg_rows[e * _NEG + r,
                                               pl.ds(k * _LANES, _LANES)]
                y_v[1 + r, e, pl.ds(0, _LANES)] = v
            return carry2

        lax.fori_loop(0, _CH, elem, 0)
        for r in range(_T):
            pltpu.sync_copy(
                y_v.at[r],
                y_hbm.at[pl.ds(pl.multiple_of(r * _B + cb, 8), _CH)])
        return carry

    lax.fori_loop(0, _NCHUNK, chunk_body, 0)


def _loss_tc_body(y2_ref, lens_ref, o_ref):
    y2 = y2_ref[...]
    seg = (lax.broadcasted_iota(jnp.int32, (_Y2COLS, _LANES), 0) // _LANES
           == lax.broadcasted_iota(jnp.int32, (_Y2COLS, _LANES), 1))
    x = jnp.dot(y2, seg.astype(jnp.float32),
                preferred_element_type=jnp.float32)
    x3 = x.reshape(_T, _BROWS, _LANES) / lens_ref[...][None, :, :]
    sgn = jnp.where(
        lax.broadcasted_iota(jnp.int32, (_T, _BROWS, _LANES), 0) == 0,
        1.0, -1.0)
    terms = -jax.nn.log_sigmoid(sgn * jnp.clip(x3, -10.0, 10.0))
    o_ref[...] = jnp.sum(terms)[None, None]


@jax.jit
def _cbow(ctx_flat, word_idx, neg_flat, lens2, emb0_weight, emb1_weight):
    mesh = plsc.VectorSubcoreMesh(core_axis_name="c", subcore_axis_name="s")
    y = pl.kernel(
        _cbow_sc_body,
        mesh=mesh,
        compiler_params=pltpu.CompilerParams(use_tc_tiling_on_sc=False),
        out_type=jax.ShapeDtypeStruct((_YROWS, _LANES), jnp.float32),
        scratch_types=[
            pltpu.VMEM((_BPW * _L,), jnp.int32),
            pltpu.VMEM((_BPW * _NEG,), jnp.int32),
            pltpu.VMEM((_BPW,), jnp.int32),
            pltpu.VMEM((_CH * _L, _DIM), jnp.float32),
            pltpu.VMEM((_CH, _DIM), jnp.float32),
            pltpu.VMEM((_CH * _NEG, _DIM), jnp.float32),
            pltpu.VMEM((_T, _CH, _LANES), jnp.float32),
            pltpu.SemaphoreType.DMA,
        ],
    )(ctx_flat, word_idx, neg_flat, emb0_weight, emb1_weight)
    o = pl.pallas_call(
        _loss_tc_body,
        out_shape=jax.ShapeDtypeStruct((1, 1), jnp.float32),
    )(y.reshape(_Y2ROWS, _Y2COLS), lens2)
    return o[0, 0]


def kernel(word_idx, ctx_inds, ctx_lens, neg_inds, emb0_weight, emb1_weight):
    lens2 = ctx_lens.astype(jnp.float32).reshape(_BROWS, _LANES)
    return _cbow(ctx_inds.astype(jnp.int32).reshape(-1),
                 word_idx.astype(jnp.int32),
                 neg_inds.astype(jnp.int32).reshape(-1), lens2,
                 emb0_weight, emb1_weight)
